# Initial kernel scaffold; baseline (speedup 1.0000x reference)
import jax
import jax.numpy as jnp
from jax.experimental import pallas as pl


def kernel(x, coefficients_vect, zero_knot_indexes, grid):
    raise NotImplementedError("write your pallas kernel here")


# speedup vs baseline: 1215.8165x; 1215.8165x over previous
import functools

import jax
import jax.numpy as jnp
from jax import lax
from jax.experimental import pallas as pl
from jax.experimental.pallas import tpu as pltpu
from jax.experimental.pallas import tpu_sc as plsc

SIZE = 51
NUM_ACT = 96
SLAB = 224 * 224
NSLAB = 8 * NUM_ACT
NC, NS = 2, 16
NW = NC * NS
SLABS_PER_W = NSLAB // NW
CHUNKS_PER_SLAB = 2
CHUNK = SLAB // CHUNKS_PER_SLAB
NCHUNK_W = SLABS_PER_W * CHUNKS_PER_SLAB
ELEMS_PER_W = SLABS_PER_W * SLAB
TOTAL = NSLAB * SLAB
TABLE = NUM_ACT * SIZE


def _body(x_hbm, ctab_hbm, dtab_hbm, zki_hbm, invg_hbm, out_hbm,
          ctab_v, dtab_v, zki_v, invg_v, xb0, xb1, ob0, ob1,
          in_sem0, in_sem1, out_sem0, out_sem1):
    wid = lax.axis_index("s") * NC + lax.axis_index("c")

    pltpu.sync_copy(ctab_hbm, ctab_v)
    pltpu.sync_copy(dtab_hbm, dtab_v)
    pltpu.sync_copy(zki_hbm, zki_v)
    pltpu.sync_copy(invg_hbm, invg_v)
    invg = invg_v[...]
    base_e = wid * ELEMS_PER_W

    def start_in(i, xb, sem):
        pltpu.async_copy(x_hbm.at[pl.ds(base_e + i * CHUNK, CHUNK)], xb, sem)

    def wait_in(xb, sem):
        pltpu.make_async_copy(x_hbm.at[pl.ds(0, CHUNK)], xb, sem).wait()

    def start_out(i, ob, sem):
        pltpu.async_copy(ob, out_hbm.at[pl.ds(base_e + i * CHUNK, CHUNK)], sem)

    def wait_out(ob, sem):
        pltpu.make_async_copy(ob, out_hbm.at[pl.ds(0, CHUNK)], sem).wait()

    def make_bvec(i):
        slab = wid * SLABS_PER_W + lax.div(i, 2)
        c = lax.rem(slab, NUM_ACT)
        zk = plsc.load_gather(zki_v, [jnp.broadcast_to(c, (16,))])
        return zk.astype(jnp.int32) - 25

    def compute(xb, ob, bvec):
        @plsc.parallel_loop(0, CHUNK, step=16, unroll=4)
        def _(o):
            v = xb[pl.ds(o, 16)]
            t = v * invg + jnp.float32(25.0)
            t = jnp.minimum(jnp.maximum(t, jnp.float32(0.0)), jnp.float32(49.0))
            j = t.astype(jnp.int32)
            frac = t - j.astype(jnp.float32)
            gi = j + bvec
            cv = plsc.load_gather(ctab_v, [gi])
            dv = plsc.load_gather(dtab_v, [gi])
            ob[pl.ds(o, 16)] = cv + frac * dv

    bufs = ((xb0, ob0, in_sem0, out_sem0), (xb1, ob1, in_sem1, out_sem1))

    start_in(0, xb0, in_sem0)
    start_in(1, xb1, in_sem1)
    for b in range(2):
        xb, ob, isem, osem = bufs[b]
        i = jnp.int32(b)
        wait_in(xb, isem)
        compute(xb, ob, make_bvec(i))
        start_out(i, ob, osem)
        start_in(i + 2, xb, isem)

    def loop_body(p, carry):
        i0 = p * 2
        for b in range(2):
            xb, ob, isem, osem = bufs[b]
            i = i0 + b
            wait_in(xb, isem)
            wait_out(ob, osem)
            compute(xb, ob, make_bvec(i))
            start_out(i, ob, osem)
            start_in(i + 2, xb, isem)
        return carry

    lax.fori_loop(1, NCHUNK_W // 2 - 1, loop_body, jnp.int32(0))

    for b in range(2):
        xb, ob, isem, osem = bufs[b]
        i = jnp.int32(NCHUNK_W - 2 + b)
        wait_in(xb, isem)
        wait_out(ob, osem)
        compute(xb, ob, make_bvec(i))
        start_out(i, ob, osem)
    for b in range(2):
        xb, ob, isem, osem = bufs[b]
        wait_out(ob, osem)


@jax.jit
def kernel(x, coefficients_vect, zero_knot_indexes, grid):
    ctab = coefficients_vect.astype(jnp.float32)
    dtab = jnp.concatenate([ctab[1:] - ctab[:-1], jnp.zeros((1,), jnp.float32)])
    invg = jnp.broadcast_to(jnp.float32(1.0) / grid[0].astype(jnp.float32), (16,))
    x_flat = x.reshape(TOTAL)

    run = pl.kernel(
        _body,
        out_type=jax.ShapeDtypeStruct((TOTAL,), jnp.float32),
        mesh=plsc.VectorSubcoreMesh(
            core_axis_name="c", subcore_axis_name="s",
            num_cores=NC, num_subcores=NS),
        compiler_params=pltpu.CompilerParams(needs_layout_passes=False),
        scratch_types=[
            pltpu.VMEM((TABLE,), jnp.float32),
            pltpu.VMEM((TABLE,), jnp.float32),
            pltpu.VMEM((128,), jnp.float32),
            pltpu.VMEM((16,), jnp.float32),
            pltpu.VMEM((CHUNK,), jnp.float32),
            pltpu.VMEM((CHUNK,), jnp.float32),
            pltpu.VMEM((CHUNK,), jnp.float32),
            pltpu.VMEM((CHUNK,), jnp.float32),
            pltpu.SemaphoreType.DMA,
            pltpu.SemaphoreType.DMA,
            pltpu.SemaphoreType.DMA,
            pltpu.SemaphoreType.DMA,
        ],
    )
    zki_pad = jnp.concatenate(
        [zero_knot_indexes.astype(jnp.float32),
         jnp.zeros((128 - NUM_ACT,), jnp.float32)])
    out_flat = run(x_flat, ctab, dtab, zki_pad, invg)
    return out_flat.reshape(x.shape)


# speedup vs baseline: 1295.1683x; 1.0653x over previous
import functools

import jax
import jax.numpy as jnp
from jax import lax
from jax.experimental import pallas as pl
from jax.experimental.pallas import tpu as pltpu
from jax.experimental.pallas import tpu_sc as plsc

SIZE = 51
NUM_ACT = 96
SLAB = 224 * 224
NSLAB = 8 * NUM_ACT
NC, NS = 2, 16
NW = NC * NS
SLABS_PER_W = NSLAB // NW
CHUNKS_PER_SLAB = 2
CHUNK = SLAB // CHUNKS_PER_SLAB
NCHUNK_W = SLABS_PER_W * CHUNKS_PER_SLAB
ELEMS_PER_W = SLABS_PER_W * SLAB
TOTAL = NSLAB * SLAB
TABLE = NUM_ACT * SIZE
CSTRIDE = 56
TABLE_PAD = NUM_ACT * CSTRIDE


def _body(x_hbm, ctab_hbm, dtab_hbm, invg_hbm, out_hbm,
          ctab_v, dtab_v, invg_v, xb0, xb1, ob0, ob1,
          in_sem0, in_sem1, out_sem0, out_sem1):
    wid = lax.axis_index("s") * NC + lax.axis_index("c")

    pltpu.sync_copy(ctab_hbm, ctab_v)
    pltpu.sync_copy(dtab_hbm, dtab_v)
    pltpu.sync_copy(invg_hbm, invg_v)
    invg = invg_v[...]
    base_e = wid * ELEMS_PER_W

    def start_in(i, xb, sem):
        pltpu.async_copy(x_hbm.at[pl.ds(base_e + i * CHUNK, CHUNK)], xb, sem)

    def wait_in(xb, sem):
        pltpu.make_async_copy(x_hbm.at[pl.ds(0, CHUNK)], xb, sem).wait()

    def start_out(i, ob, sem):
        pltpu.async_copy(ob, out_hbm.at[pl.ds(base_e + i * CHUNK, CHUNK)], sem)

    def wait_out(ob, sem):
        pltpu.make_async_copy(ob, out_hbm.at[pl.ds(0, CHUNK)], sem).wait()

    def make_base(i):
        slab = wid * SLABS_PER_W + lax.div(i, 2)
        c = lax.rem(slab, NUM_ACT)
        return c * CSTRIDE

    def compute(xb, ob, base):
        ctab_sl = ctab_v.at[pl.ds(base, CSTRIDE)]
        dtab_sl = dtab_v.at[pl.ds(base, CSTRIDE)]

        @plsc.parallel_loop(0, CHUNK, step=16, unroll=8)
        def _(o):
            v = xb[pl.ds(o, 16)]
            t = v * invg + jnp.float32(25.0)
            t = jnp.minimum(jnp.maximum(t, jnp.float32(0.0)), jnp.float32(49.0))
            j = t.astype(jnp.int32)
            frac = t - j.astype(jnp.float32)
            cv = plsc.load_gather(ctab_sl, [j])
            dv = plsc.load_gather(dtab_sl, [j])
            ob[pl.ds(o, 16)] = cv + frac * dv

    bufs = ((xb0, ob0, in_sem0, out_sem0), (xb1, ob1, in_sem1, out_sem1))

    start_in(0, xb0, in_sem0)
    start_in(1, xb1, in_sem1)
    for b in range(2):
        xb, ob, isem, osem = bufs[b]
        i = jnp.int32(b)
        wait_in(xb, isem)
        compute(xb, ob, make_base(i))
        start_out(i, ob, osem)
        start_in(i + 2, xb, isem)

    def loop_body(p, carry):
        i0 = p * 2
        for b in range(2):
            xb, ob, isem, osem = bufs[b]
            i = i0 + b
            wait_in(xb, isem)
            wait_out(ob, osem)
            compute(xb, ob, make_base(i))
            start_out(i, ob, osem)
            start_in(i + 2, xb, isem)
        return carry

    lax.fori_loop(1, NCHUNK_W // 2 - 1, loop_body, jnp.int32(0))

    for b in range(2):
        xb, ob, isem, osem = bufs[b]
        i = jnp.int32(NCHUNK_W - 2 + b)
        wait_in(xb, isem)
        wait_out(ob, osem)
        compute(xb, ob, make_base(i))
        start_out(i, ob, osem)
    for b in range(2):
        xb, ob, isem, osem = bufs[b]
        wait_out(ob, osem)


@jax.jit
def kernel(x, coefficients_vect, zero_knot_indexes, grid):
    del zero_knot_indexes
    cv2 = coefficients_vect.astype(jnp.float32).reshape(NUM_ACT, SIZE)
    pad = jnp.zeros((NUM_ACT, CSTRIDE - SIZE), jnp.float32)
    ctab = jnp.concatenate([cv2, pad], axis=1).reshape(TABLE_PAD)
    dv2 = jnp.concatenate(
        [cv2[:, 1:] - cv2[:, :-1], jnp.zeros((NUM_ACT, 1), jnp.float32)], axis=1)
    dtab = jnp.concatenate([dv2, pad], axis=1).reshape(TABLE_PAD)
    invg = jnp.broadcast_to(jnp.float32(1.0) / grid[0].astype(jnp.float32), (16,))
    x_flat = x.reshape(TOTAL)

    run = pl.kernel(
        _body,
        out_type=jax.ShapeDtypeStruct((TOTAL,), jnp.float32),
        mesh=plsc.VectorSubcoreMesh(
            core_axis_name="c", subcore_axis_name="s",
            num_cores=NC, num_subcores=NS),
        compiler_params=pltpu.CompilerParams(needs_layout_passes=False),
        scratch_types=[
            pltpu.VMEM((TABLE_PAD,), jnp.float32),
            pltpu.VMEM((TABLE_PAD,), jnp.float32),
            pltpu.VMEM((16,), jnp.float32),
            pltpu.VMEM((CHUNK,), jnp.float32),
            pltpu.VMEM((CHUNK,), jnp.float32),
            pltpu.VMEM((CHUNK,), jnp.float32),
            pltpu.VMEM((CHUNK,), jnp.float32),
            pltpu.SemaphoreType.DMA,
            pltpu.SemaphoreType.DMA,
            pltpu.SemaphoreType.DMA,
            pltpu.SemaphoreType.DMA,
        ],
    )
    out_flat = run(x_flat, ctab, dtab, invg)
    return out_flat.reshape(x.shape)


# speedup vs baseline: 3440.3864x; 2.6563x over previous
import functools

import jax
import jax.numpy as jnp
from jax import lax
from jax.experimental import pallas as pl
from jax.experimental.pallas import tpu as pltpu
from jax.experimental.pallas import tpu_sc as plsc

SIZE = 51
NUM_ACT = 96
SLAB = 224 * 224
NSLAB = 8 * NUM_ACT
NC, NS = 2, 16
NW = NC * NS
SLABS_PER_W = NSLAB // NW
CHUNKS_PER_SLAB = 2
CHUNK = SLAB // CHUNKS_PER_SLAB
NCHUNK_W = SLABS_PER_W * CHUNKS_PER_SLAB
ELEMS_PER_W = SLABS_PER_W * SLAB
TOTAL = NSLAB * SLAB
TABLE = NUM_ACT * SIZE
CSTRIDE = 56
TABLE_PAD = NUM_ACT * CSTRIDE


def _body(x_hbm, ctab_hbm, dtab_hbm, invg_hbm, out_hbm,
          ctab_v, dtab_v, invg_v, xb0, xb1, ob0, ob1,
          in_sem0, in_sem1, out_sem0, out_sem1):
    wid = lax.axis_index("s") * NC + lax.axis_index("c")

    pltpu.sync_copy(ctab_hbm, ctab_v)
    pltpu.sync_copy(dtab_hbm, dtab_v)
    pltpu.sync_copy(invg_hbm, invg_v)
    invg = invg_v[...]
    base_e = wid * ELEMS_PER_W

    def start_in(i, xb, sem):
        pltpu.async_copy(x_hbm.at[pl.ds(base_e + i * CHUNK, CHUNK)], xb, sem)

    def wait_in(xb, sem):
        pltpu.make_async_copy(x_hbm.at[pl.ds(0, CHUNK)], xb, sem).wait()

    def start_out(i, ob, sem):
        pltpu.async_copy(ob, out_hbm.at[pl.ds(base_e + i * CHUNK, CHUNK)], sem)

    def wait_out(ob, sem):
        pltpu.make_async_copy(ob, out_hbm.at[pl.ds(0, CHUNK)], sem).wait()

    def make_base(i):
        slab = wid * SLABS_PER_W + lax.div(i, 2)
        c = lax.rem(slab, NUM_ACT)
        return c * CSTRIDE

    def compute(xb, ob, base):
        ctab_sl = ctab_v.at[pl.ds(base, CSTRIDE)]
        dtab_sl = dtab_v.at[pl.ds(base, CSTRIDE)]

        @plsc.parallel_loop(0, CHUNK, step=16, unroll=8)
        def _(o):
            v = xb[pl.ds(o, 16)]
            t = v * invg + jnp.float32(25.0)
            t = jnp.minimum(jnp.maximum(t, jnp.float32(0.0)), jnp.float32(49.0))
            j = t.astype(jnp.int32)
            frac = t - j.astype(jnp.float32)
            cv = plsc.load_gather(ctab_sl, [j])
            dv = plsc.load_gather(dtab_sl, [j])
            ob[pl.ds(o, 16)] = cv + frac * dv

    bufs = ((xb0, ob0, in_sem0, out_sem0), (xb1, ob1, in_sem1, out_sem1))

    start_in(0, xb0, in_sem0)
    start_in(1, xb1, in_sem1)
    for b in range(2):
        xb, ob, isem, osem = bufs[b]
        i = jnp.int32(b)
        wait_in(xb, isem)
        compute(xb, ob, make_base(i))
        start_out(i, ob, osem)
        start_in(i + 2, xb, isem)

    def loop_body(p, carry):
        i0 = p * 2
        for b in range(2):
            xb, ob, isem, osem = bufs[b]
            i = i0 + b
            wait_in(xb, isem)
            wait_out(ob, osem)
            compute(xb, ob, make_base(i))
            start_out(i, ob, osem)
            start_in(i + 2, xb, isem)
        return carry

    lax.fori_loop(1, NCHUNK_W // 2 - 1, loop_body, jnp.int32(0))

    for b in range(2):
        xb, ob, isem, osem = bufs[b]
        i = jnp.int32(NCHUNK_W - 2 + b)
        wait_in(xb, isem)
        wait_out(ob, osem)
        compute(xb, ob, make_base(i))
        start_out(i, ob, osem)
    for b in range(2):
        xb, ob, isem, osem = bufs[b]
        wait_out(ob, osem)


def _body_tiled(x_hbm, ctab_hbm, dtab_hbm, invg_hbm, out_hbm,
                ctab_v, dtab_v, invg_v, xb0, xb1, ob0, ob1,
                in_sem0, in_sem1, out_sem0, out_sem1):
    wid = lax.axis_index("s") * NC + lax.axis_index("c")

    pltpu.sync_copy(ctab_hbm, ctab_v)
    pltpu.sync_copy(dtab_hbm, dtab_v)
    pltpu.sync_copy(invg_hbm, invg_v)
    invg = invg_v[...]
    slab0 = wid * SLABS_PER_W

    def start_in(s, xb, sem, c0, w):
        pltpu.async_copy(x_hbm.at[slab0 + s, :, pl.ds(c0, w)], xb, sem)

    def wait_in(xb, sem, c0, w):
        pltpu.make_async_copy(x_hbm.at[0, :, pl.ds(c0, w)], xb, sem).wait()

    def start_out(s, ob, sem, c0, w):
        pltpu.async_copy(ob, out_hbm.at[slab0 + s, :, pl.ds(c0, w)], sem)

    def wait_out(ob, sem, c0, w):
        pltpu.make_async_copy(ob, out_hbm.at[0, :, pl.ds(c0, w)], sem).wait()

    def make_base(s):
        c = lax.rem(slab0 + s, NUM_ACT)
        return c * CSTRIDE

    def compute(xb, ob, base, w):
        ctab_sl = ctab_v.at[pl.ds(base, CSTRIDE)]
        dtab_sl = dtab_v.at[pl.ds(base, CSTRIDE)]

        @plsc.parallel_loop(0, 224, step=1, unroll=2)
        def _(r):
            for k in range(w // 16):
                v = xb[r, pl.ds(k * 16, 16)]
                t = v * invg + jnp.float32(25.0)
                t = jnp.minimum(jnp.maximum(t, jnp.float32(0.0)),
                                jnp.float32(49.0))
                j = t.astype(jnp.int32)
                frac = t - j.astype(jnp.float32)
                cv = plsc.load_gather(ctab_sl, [j])
                dv = plsc.load_gather(dtab_sl, [j])
                ob[r, pl.ds(k * 16, 16)] = cv + frac * dv

    lanes = ((xb0, ob0, in_sem0, out_sem0, 0, 128),
             (xb1, ob1, in_sem1, out_sem1, 128, 96))

    for xb, ob, isem, osem, c0, w in lanes:
        start_in(jnp.int32(0), xb, isem, c0, w)
    for xb, ob, isem, osem, c0, w in lanes:
        s = jnp.int32(0)
        wait_in(xb, isem, c0, w)
        compute(xb, ob, make_base(s), w)
        start_out(s, ob, osem, c0, w)
        start_in(s + 1, xb, isem, c0, w)

    def loop_body(s, carry):
        for xb, ob, isem, osem, c0, w in lanes:
            wait_in(xb, isem, c0, w)
            wait_out(ob, osem, c0, w)
            compute(xb, ob, make_base(s), w)
            start_out(s, ob, osem, c0, w)
            start_in(s + 1, xb, isem, c0, w)
        return carry

    lax.fori_loop(1, SLABS_PER_W - 1, loop_body, jnp.int32(0))

    s_last = jnp.int32(SLABS_PER_W - 1)
    for xb, ob, isem, osem, c0, w in lanes:
        wait_in(xb, isem, c0, w)
        wait_out(ob, osem, c0, w)
        compute(xb, ob, make_base(s_last), w)
        start_out(s_last, ob, osem, c0, w)
    for xb, ob, isem, osem, c0, w in lanes:
        wait_out(ob, osem, c0, w)


@jax.jit
def kernel_sc_tiled(x, coefficients_vect, zero_knot_indexes, grid):
    del zero_knot_indexes
    cv2 = coefficients_vect.astype(jnp.float32).reshape(NUM_ACT, SIZE)
    pad = jnp.zeros((NUM_ACT, CSTRIDE - SIZE), jnp.float32)
    ctab = jnp.concatenate([cv2, pad], axis=1).reshape(TABLE_PAD)
    dv2 = jnp.concatenate(
        [cv2[:, 1:] - cv2[:, :-1], jnp.zeros((NUM_ACT, 1), jnp.float32)], axis=1)
    dtab = jnp.concatenate([dv2, pad], axis=1).reshape(TABLE_PAD)
    invg = jnp.broadcast_to(jnp.float32(1.0) / grid[0].astype(jnp.float32), (16,))
    x3 = x.reshape(NSLAB, 224, 224)

    run = pl.kernel(
        _body_tiled,
        out_type=jax.ShapeDtypeStruct((NSLAB, 224, 224), jnp.float32),
        mesh=plsc.VectorSubcoreMesh(
            core_axis_name="c", subcore_axis_name="s",
            num_cores=NC, num_subcores=NS),
        compiler_params=pltpu.CompilerParams(
            needs_layout_passes=False, use_tc_tiling_on_sc=True),
        scratch_types=[
            pltpu.VMEM((TABLE_PAD,), jnp.float32),
            pltpu.VMEM((TABLE_PAD,), jnp.float32),
            pltpu.VMEM((16,), jnp.float32),
            pltpu.VMEM((224, 128), jnp.float32),
            pltpu.VMEM((224, 96), jnp.float32),
            pltpu.VMEM((224, 128), jnp.float32),
            pltpu.VMEM((224, 96), jnp.float32),
            pltpu.SemaphoreType.DMA,
            pltpu.SemaphoreType.DMA,
            pltpu.SemaphoreType.DMA,
            pltpu.SemaphoreType.DMA,
        ],
    )
    out3 = run(x3, ctab, dtab, invg)
    return out3.reshape(x.shape)


def _tc_body(x_ref, ctab_ref, dtab_ref, invg_ref, o_ref):
    x = x_ref[0]
    invg = invg_ref[0, 0]
    t = x * invg + jnp.float32(25.0)
    t = jnp.minimum(jnp.maximum(t, jnp.float32(0.0)), jnp.float32(49.0))
    j = t.astype(jnp.int32)
    frac = t - j.astype(jnp.float32)
    ctab = jnp.broadcast_to(ctab_ref[0], (224, 128))
    dtab = jnp.broadcast_to(dtab_ref[0], (224, 128))
    cv = jnp.take_along_axis(ctab, j, axis=1)
    dv = jnp.take_along_axis(dtab, j, axis=1)
    o_ref[0] = cv + frac * dv


def _run_tc(x3, ctab_bc, dtab_bc, invg_s):
    return pl.pallas_call(
        _tc_body,
        out_shape=jax.ShapeDtypeStruct(x3.shape, jnp.float32),
        in_specs=[
            pl.BlockSpec((1, 224, 128), lambda i, cb: (i, 0, cb)),
            pl.BlockSpec((1, 1, 128), lambda i, cb: (lax.rem(i, NUM_ACT), 0, 0)),
            pl.BlockSpec((1, 1, 128), lambda i, cb: (lax.rem(i, NUM_ACT), 0, 0)),
            pl.BlockSpec(memory_space=pltpu.SMEM),
        ],
        out_specs=pl.BlockSpec((1, 224, 128), lambda i, cb: (i, 0, cb)),
        grid=(NSLAB, 2),
    )(x3, ctab_bc, dtab_bc, invg_s)


@jax.jit
def kernel_tc(x, coefficients_vect, zero_knot_indexes, grid):
    del zero_knot_indexes
    cv2 = coefficients_vect.astype(jnp.float32).reshape(NUM_ACT, SIZE)
    pad = jnp.zeros((NUM_ACT, 128 - SIZE), jnp.float32)
    ctab_r = jnp.concatenate([cv2, pad], axis=1)
    dv2 = jnp.concatenate(
        [cv2[:, 1:] - cv2[:, :-1], jnp.zeros((NUM_ACT, 1), jnp.float32)], axis=1)
    dtab_r = jnp.concatenate([dv2, pad], axis=1)
    ctab_bc = ctab_r[:, None, :]
    dtab_bc = dtab_r[:, None, :]
    invg_s = (jnp.float32(1.0) / grid[0].astype(jnp.float32)).reshape(1, 1)
    x3 = x.reshape(NSLAB, 224, 224)
    out = _run_tc(x3, ctab_bc, dtab_bc, invg_s)
    return out.reshape(x.shape)


@jax.jit
def kernel(x, coefficients_vect, zero_knot_indexes, grid):
    del zero_knot_indexes
    cv2 = coefficients_vect.astype(jnp.float32).reshape(NUM_ACT, SIZE)
    pad = jnp.zeros((NUM_ACT, CSTRIDE - SIZE), jnp.float32)
    ctab = jnp.concatenate([cv2, pad], axis=1).reshape(TABLE_PAD)
    dv2 = jnp.concatenate(
        [cv2[:, 1:] - cv2[:, :-1], jnp.zeros((NUM_ACT, 1), jnp.float32)], axis=1)
    dtab = jnp.concatenate([dv2, pad], axis=1).reshape(TABLE_PAD)
    invg = jnp.broadcast_to(jnp.float32(1.0) / grid[0].astype(jnp.float32), (16,))
    x_flat = x.reshape(TOTAL)

    run = pl.kernel(
        _body,
        out_type=jax.ShapeDtypeStruct((TOTAL,), jnp.float32),
        mesh=plsc.VectorSubcoreMesh(
            core_axis_name="c", subcore_axis_name="s",
            num_cores=NC, num_subcores=NS),
        compiler_params=pltpu.CompilerParams(needs_layout_passes=False),
        scratch_types=[
            pltpu.VMEM((TABLE_PAD,), jnp.float32),
            pltpu.VMEM((TABLE_PAD,), jnp.float32),
            pltpu.VMEM((16,), jnp.float32),
            pltpu.VMEM((CHUNK,), jnp.float32),
            pltpu.VMEM((CHUNK,), jnp.float32),
            pltpu.VMEM((CHUNK,), jnp.float32),
            pltpu.VMEM((CHUNK,), jnp.float32),
            pltpu.SemaphoreType.DMA,
            pltpu.SemaphoreType.DMA,
            pltpu.SemaphoreType.DMA,
            pltpu.SemaphoreType.DMA,
        ],
    )
    out_flat = run(x_flat, ctab, dtab, invg)
    return out_flat.reshape(x.shape)


_kernel_sc_linear = kernel
kernel = kernel_sc_tiled
